# baseline (speedup 1.0000x reference)
import jax
import jax.numpy as jnp
from jax import lax
from jax.experimental import pallas as pl
from jax.experimental.pallas import tpu as pltpu
from jax.experimental.pallas import tpu_sc as plsc

TOKENS = 2048
D_MODEL = 768
D_EXPERT = 384
E = 8
TOP_K = 2

BLK = 256
NB = TOKENS * TOP_K // BLK + E
CAP = NB * BLK

NC = 2
NS = 16
NW = NC * NS
TPW = TOKENS // NW
RPW = CAP // NW


def _router_body(x_ref, c_ref, b_ref,
                 fi_ref, pi_ref, bal_ref, dl_ref, bm_ref, bs_ref,
                 p1_ref, p2_ref, g1_ref, g2_ref, be_ref):
    x = x_ref[...]
    c = c_ref[...]
    b = b_ref[...]
    logits = lax.dot_general(
        x, c, (((1,), (1,)), ((), ())),
        preferred_element_type=jnp.float32) + b
    scores = jax.nn.sigmoid(logits)
    e_iota = lax.broadcasted_iota(jnp.int32, scores.shape, 1)
    m1 = jnp.max(scores, axis=1, keepdims=True)
    idx1 = jnp.min(jnp.where(scores == m1, e_iota, E), axis=1, keepdims=True)
    oh1 = (e_iota == idx1).astype(jnp.float32)
    masked = jnp.where(e_iota == idx1, -jnp.inf, scores)
    m2 = jnp.max(masked, axis=1, keepdims=True)
    idx2 = jnp.min(jnp.where(masked == m2, e_iota, E), axis=1, keepdims=True)
    oh2 = (e_iota == idx2).astype(jnp.float32)
    denom = jnp.clip(m1 + m2, 1e-9, None)
    g1 = m1 / denom
    g2 = m2 / denom
    n = x.shape[0]

    sel = oh1 + oh2
    counts = None
    inc = sel
    d = 1
    while d < n:
        inc = inc + jnp.concatenate(
            [jnp.zeros((d, E), jnp.float32), inc[:-d, :]], axis=0)
        d *= 2
    counts = inc[n - 1:n, :]
    fi = counts / (n * TOP_K)
    pi = jnp.sum(g1 * oh1 + g2 * oh2, axis=0, keepdims=True) / n
    fi_ref[...] = fi
    pi_ref[...] = pi
    bal_ref[...] = jnp.sum(fi * pi, keepdims=True).reshape(1, 1)
    dl_ref[...] = jnp.sum(fi, keepdims=True).reshape(1, 1)
    bm = jnp.mean(b)
    bm_ref[...] = bm.reshape(1, 1)
    bs_ref[...] = jnp.sqrt(jnp.sum((b - bm) ** 2) / (E - 1)).reshape(1, 1)

    excl = inc - sel
    counts_i = counts.astype(jnp.int32)
    pc = (((counts_i + (BLK - 1)) // BLK) * BLK).astype(jnp.float32)
    tri = (lax.broadcasted_iota(jnp.int32, (E, E), 0) <
           lax.broadcasted_iota(jnp.int32, (E, E), 1)).astype(jnp.float32)
    off = lax.dot_general(pc, tri, (((1,), (0,)), ((), ())),
                          preferred_element_type=jnp.float32)
    ends = off + pc
    rank1 = jnp.sum(oh1 * excl, axis=1, keepdims=True)
    rank2 = jnp.sum(oh2 * excl, axis=1, keepdims=True)
    base1 = jnp.sum(oh1 * off, axis=1, keepdims=True)
    base2 = jnp.sum(oh2 * off, axis=1, keepdims=True)
    p1_ref[...] = (base1 + rank1).astype(jnp.int32)
    p2_ref[...] = (base2 + rank2).astype(jnp.int32)
    g1_ref[...] = g1
    g2_ref[...] = g2
    bstart = (lax.broadcasted_iota(jnp.int32, (NB, E), 0) * BLK).astype(
        jnp.float32)
    nseg_done = jnp.sum((bstart >= ends).astype(jnp.int32), axis=1,
                        keepdims=True)
    be_ref[...] = jnp.minimum(nseg_done, E - 1)


def _dispatch_body(x_hbm, p1_hbm, p2_hbm, g1_hbm, g2_hbm,
                   xs_hbm, gs_hbm,
                   zi, zf, tok_v, idx_v, val_v, toksl, gsl, rows_v, sem,
                   sh_tok, sh_g):
    cid = lax.axis_index("c")
    sid = lax.axis_index("s")

    z16i = jnp.zeros((16,), jnp.int32)
    z16f = jnp.zeros((16,), jnp.float32)
    for j in range(CAP // NS // 16):
        zi[pl.ds(j * 16, 16)] = z16i
        zf[pl.ds(j * 16, 16)] = z16f
    pltpu.sync_copy(zi, sh_tok.at[pl.ds(sid * (CAP // NS), CAP // NS)])
    pltpu.sync_copy(zf, sh_g.at[pl.ds(sid * (CAP // NS), CAP // NS)])
    plsc.subcore_barrier()

    tpt = TOKENS // NS
    for j in range(tpt // 16):
        tok_v[pl.ds(j * 16, 16)] = (
            sid * tpt + j * 16 + jnp.arange(16, dtype=jnp.int32))
    for p_hbm, g_hbm in ((p1_hbm, g1_hbm), (p2_hbm, g2_hbm)):
        pltpu.sync_copy(p_hbm.at[pl.ds(sid * tpt, tpt)], idx_v)
        pltpu.sync_copy(g_hbm.at[pl.ds(sid * tpt, tpt)], val_v)
        pltpu.sync_copy(tok_v, sh_tok.at[idx_v], add=True)
        pltpu.sync_copy(val_v, sh_g.at[idx_v], add=True)
    plsc.subcore_barrier()

    wid = sid * NC + cid
    base = wid * RPW
    pltpu.sync_copy(sh_g.at[pl.ds(base, RPW)], gsl)
    pltpu.sync_copy(gsl, gs_hbm.at[pl.ds(base, RPW)])
    pltpu.sync_copy(sh_tok.at[pl.ds(base, RPW)], toksl)
    half = RPW // 2
    for c2 in range(2):
        cp = pltpu.async_copy(
            x_hbm.at[toksl.at[pl.ds(c2 * half, half)]], rows_v, sem)
        cp.wait()
        pltpu.sync_copy(rows_v, xs_hbm.at[pl.ds(base + c2 * half, half)])


def _expert_body(be_ref, xs_ref, w1_ref, w3_ref, w2_ref, gs_ref, scale_ref,
                 eo_ref):
    xs = xs_ref[...]
    h1 = jnp.dot(xs, w1_ref[0], preferred_element_type=jnp.float32)
    h3 = jnp.dot(xs, w3_ref[0], preferred_element_type=jnp.float32)
    h = h1 * jax.nn.sigmoid(h1) * h3
    eo = jnp.dot(h, w2_ref[0], preferred_element_type=jnp.float32)
    eo_ref[...] = eo * gs_ref[...] * scale_ref[0, 0]


def _combine_body(eo_hbm, p1_hbm, p2_hbm, out_hbm,
                  idx_a, idx_b, rows_a, rows_b, sem):
    cid = lax.axis_index("c")
    sid = lax.axis_index("s")
    wid = sid * NC + cid
    t0 = wid * TPW
    pltpu.sync_copy(p1_hbm.at[pl.ds(t0, TPW)], idx_a)
    pltpu.sync_copy(p2_hbm.at[pl.ds(t0, TPW)], idx_b)
    cpa = pltpu.async_copy(eo_hbm.at[idx_a], rows_a, sem)
    cpb = pltpu.async_copy(eo_hbm.at[idx_b], rows_b, sem)
    cpa.wait()
    cpb.wait()

    def body(i, carry):
        for j in range(D_MODEL // 16):
            s = pl.ds(j * 16, 16)
            rows_a[i, s] = rows_a[i, s] + rows_b[i, s]
        return carry

    lax.fori_loop(0, TPW, body, 0)
    pltpu.sync_copy(rows_a, out_hbm.at[pl.ds(t0, TPW)])


@jax.jit
def kernel(x, centroids, w1, w3, w2, bias, scale):
    sc_mesh = plsc.VectorSubcoreMesh(
        core_axis_name="c", subcore_axis_name="s", num_cores=NC)
    n = x.shape[0]
    b2 = bias.reshape(1, E)
    fi, pi, bal, dl, bm, bs, p1, p2, g1, g2, be = pl.pallas_call(
        _router_body,
        out_shape=(
            jax.ShapeDtypeStruct((1, E), jnp.float32),
            jax.ShapeDtypeStruct((1, E), jnp.float32),
            jax.ShapeDtypeStruct((1, 1), jnp.float32),
            jax.ShapeDtypeStruct((1, 1), jnp.float32),
            jax.ShapeDtypeStruct((1, 1), jnp.float32),
            jax.ShapeDtypeStruct((1, 1), jnp.float32),
            jax.ShapeDtypeStruct((n, 1), jnp.int32),
            jax.ShapeDtypeStruct((n, 1), jnp.int32),
            jax.ShapeDtypeStruct((n, 1), jnp.float32),
            jax.ShapeDtypeStruct((n, 1), jnp.float32),
            jax.ShapeDtypeStruct((NB, 1), jnp.int32),
        ),
    )(x, centroids, b2)

    p1f = p1.reshape(n)
    p2f = p2.reshape(n)
    g1f = g1.reshape(n)
    g2f = g2.reshape(n)

    dispatch = pl.kernel(
        _dispatch_body,
        out_type=(
            jax.ShapeDtypeStruct((CAP, D_MODEL), jnp.float32),
            jax.ShapeDtypeStruct((CAP,), jnp.float32),
        ),
        mesh=sc_mesh,
        scratch_types=[
            pltpu.VMEM((CAP // NS,), jnp.int32),
            pltpu.VMEM((CAP // NS,), jnp.float32),
            pltpu.VMEM((TOKENS // NS,), jnp.int32),
            pltpu.VMEM((TOKENS // NS,), jnp.int32),
            pltpu.VMEM((TOKENS // NS,), jnp.float32),
            pltpu.VMEM((RPW,), jnp.int32),
            pltpu.VMEM((RPW,), jnp.float32),
            pltpu.VMEM((RPW // 2, D_MODEL), jnp.float32),
            pltpu.SemaphoreType.DMA,
            pltpu.VMEM_SHARED((CAP,), jnp.int32),
            pltpu.VMEM_SHARED((CAP,), jnp.float32),
        ],
    )
    xs, gs = dispatch(x, p1f, p2f, g1f, g2f)

    eo = pl.pallas_call(
        _expert_body,
        grid_spec=pltpu.PrefetchScalarGridSpec(
            num_scalar_prefetch=1,
            grid=(NB,),
            in_specs=[
                pl.BlockSpec((BLK, D_MODEL), lambda b, be: (b, 0)),
                pl.BlockSpec((1, D_MODEL, D_EXPERT), lambda b, be: (be[b], 0, 0)),
                pl.BlockSpec((1, D_MODEL, D_EXPERT), lambda b, be: (be[b], 0, 0)),
                pl.BlockSpec((1, D_EXPERT, D_MODEL), lambda b, be: (be[b], 0, 0)),
                pl.BlockSpec((BLK, 1), lambda b, be: (b, 0)),
                pl.BlockSpec((1, 1), lambda b, be: (0, 0)),
            ],
            out_specs=pl.BlockSpec((BLK, D_MODEL), lambda b, be: (b, 0)),
        ),
        out_shape=jax.ShapeDtypeStruct((CAP, D_MODEL), jnp.float32),
        compiler_params=pltpu.CompilerParams(
            dimension_semantics=("arbitrary",),
        ),
    )(be.reshape(NB), xs, w1, w3, w2, gs.reshape(CAP, 1), scale.reshape(1, 1))

    combine = pl.kernel(
        _combine_body,
        out_type=jax.ShapeDtypeStruct((n, D_MODEL), jnp.float32),
        mesh=sc_mesh,
        scratch_types=[
            pltpu.VMEM((TPW,), jnp.int32),
            pltpu.VMEM((TPW,), jnp.int32),
            pltpu.VMEM((TPW, D_MODEL), jnp.float32),
            pltpu.VMEM((TPW, D_MODEL), jnp.float32),
            pltpu.SemaphoreType.DMA,
        ],
    )
    out = combine(eo, p1f, p2f)

    return (out, fi.reshape(E), pi.reshape(E), bal.reshape(()),
            dl.reshape(1), bm.reshape(()), bs.reshape(()))


# speedup vs baseline: 2.0356x; 2.0356x over previous
import jax
import jax.numpy as jnp
from jax import lax
from jax.experimental import pallas as pl
from jax.experimental.pallas import tpu as pltpu
from jax.experimental.pallas import tpu_sc as plsc

TOKENS = 2048
D_MODEL = 768
D_EXPERT = 384
E = 8
TOP_K = 2

BLK = 256
NB = TOKENS * TOP_K // BLK + E
CAP = NB * BLK

NC = 2
NS = 16
NW = NC * NS
TPW = TOKENS // NW
RPW = CAP // NW


def _router_body(x_ref, c_ref, b_ref,
                 fi_ref, pi_ref, bal_ref, dl_ref, bm_ref, bs_ref,
                 p1_ref, p2_ref, g1_ref, g2_ref, be_ref):
    x = x_ref[...]
    c = c_ref[...]
    b = b_ref[...]
    logits = lax.dot_general(
        x, c, (((1,), (1,)), ((), ())),
        preferred_element_type=jnp.float32) + b
    scores = jax.nn.sigmoid(logits)
    e_iota = lax.broadcasted_iota(jnp.int32, scores.shape, 1)
    m1 = jnp.max(scores, axis=1, keepdims=True)
    idx1 = jnp.min(jnp.where(scores == m1, e_iota, E), axis=1, keepdims=True)
    oh1 = (e_iota == idx1).astype(jnp.float32)
    masked = jnp.where(e_iota == idx1, -jnp.inf, scores)
    m2 = jnp.max(masked, axis=1, keepdims=True)
    idx2 = jnp.min(jnp.where(masked == m2, e_iota, E), axis=1, keepdims=True)
    oh2 = (e_iota == idx2).astype(jnp.float32)
    denom = jnp.clip(m1 + m2, 1e-9, None)
    g1 = m1 / denom
    g2 = m2 / denom
    n = x.shape[0]

    sel = oh1 + oh2
    counts = None
    inc = sel
    d = 1
    while d < n:
        inc = inc + jnp.concatenate(
            [jnp.zeros((d, E), jnp.float32), inc[:-d, :]], axis=0)
        d *= 2
    counts = inc[n - 1:n, :]
    fi = counts / (n * TOP_K)
    pi = jnp.sum(g1 * oh1 + g2 * oh2, axis=0, keepdims=True) / n
    fi_ref[...] = fi
    pi_ref[...] = pi
    bal_ref[...] = jnp.sum(fi * pi, keepdims=True).reshape(1, 1)
    dl_ref[...] = jnp.sum(fi, keepdims=True).reshape(1, 1)
    bm = jnp.mean(b)
    bm_ref[...] = bm.reshape(1, 1)
    bs_ref[...] = jnp.sqrt(jnp.sum((b - bm) ** 2) / (E - 1)).reshape(1, 1)

    excl = inc - sel
    counts_i = counts.astype(jnp.int32)
    pc = (((counts_i + (BLK - 1)) // BLK) * BLK).astype(jnp.float32)
    tri = (lax.broadcasted_iota(jnp.int32, (E, E), 0) <
           lax.broadcasted_iota(jnp.int32, (E, E), 1)).astype(jnp.float32)
    off = lax.dot_general(pc, tri, (((1,), (0,)), ((), ())),
                          preferred_element_type=jnp.float32)
    ends = off + pc
    rank1 = jnp.sum(oh1 * excl, axis=1, keepdims=True)
    rank2 = jnp.sum(oh2 * excl, axis=1, keepdims=True)
    base1 = jnp.sum(oh1 * off, axis=1, keepdims=True)
    base2 = jnp.sum(oh2 * off, axis=1, keepdims=True)
    p1_ref[...] = (base1 + rank1).astype(jnp.int32)
    p2_ref[...] = (base2 + rank2).astype(jnp.int32)
    ones16 = jnp.ones((1, 16), jnp.float32)
    g1_ref[...] = g1 * ones16
    g2_ref[...] = g2 * ones16
    bstart = (lax.broadcasted_iota(jnp.int32, (NB, E), 0) * BLK).astype(
        jnp.float32)
    nseg_done = jnp.sum((bstart >= ends).astype(jnp.int32), axis=1,
                        keepdims=True)
    be_ref[...] = jnp.minimum(nseg_done, E - 1)


def _dispatch_body(x_hbm, p1_hbm, p2_hbm, xs_hbm,
                   ia, ib, rows_v, sem):
    cid = lax.axis_index("c")
    sid = lax.axis_index("s")
    wid = sid * NC + cid
    t0 = wid * TPW
    pltpu.sync_copy(p1_hbm.at[pl.ds(t0, TPW)], ia)
    pltpu.sync_copy(p2_hbm.at[pl.ds(t0, TPW)], ib)
    pltpu.sync_copy(x_hbm.at[pl.ds(t0, TPW)], rows_v)
    cpa = pltpu.async_copy(rows_v, xs_hbm.at[ia], sem)
    cpb = pltpu.async_copy(rows_v, xs_hbm.at[ib], sem)
    cpa.wait()
    cpb.wait()


def _expert_body(be_ref, xs_ref, w1_ref, w3_ref, w2_ref, scale_ref,
                 eo_ref):
    e = be_ref[pl.program_id(0)]
    xs = xs_ref[...].astype(jnp.bfloat16)
    h1 = jnp.dot(xs, w1_ref[e], preferred_element_type=jnp.float32)
    h3 = jnp.dot(xs, w3_ref[e], preferred_element_type=jnp.float32)
    h = (h1 * jax.nn.sigmoid(h1) * h3).astype(jnp.bfloat16)
    eo = jnp.dot(h, w2_ref[e], preferred_element_type=jnp.float32)
    eo_ref[...] = eo * scale_ref[0, 0]


def _combine_body(eo_hbm, p1_hbm, p2_hbm, g1_hbm, g2_hbm, out_hbm,
                  idx_a, idx_b, ga, gb, rows_a, rows_b, sem):
    cid = lax.axis_index("c")
    sid = lax.axis_index("s")
    wid = sid * NC + cid
    t0 = wid * TPW
    pltpu.sync_copy(p1_hbm.at[pl.ds(t0, TPW)], idx_a)
    pltpu.sync_copy(p2_hbm.at[pl.ds(t0, TPW)], idx_b)
    cpa = pltpu.async_copy(eo_hbm.at[idx_a], rows_a, sem)
    cpb = pltpu.async_copy(eo_hbm.at[idx_b], rows_b, sem)
    pltpu.sync_copy(g1_hbm.at[pl.ds(t0, TPW)], ga)
    pltpu.sync_copy(g2_hbm.at[pl.ds(t0, TPW)], gb)
    cpa.wait()
    cpb.wait()

    def body(i, carry):
        gav = ga[i]
        gbv = gb[i]
        for j in range(D_MODEL // 16):
            s = pl.ds(j * 16, 16)
            rows_a[i, s] = gav * rows_a[i, s] + gbv * rows_b[i, s]
        return carry

    lax.fori_loop(0, TPW, body, 0)
    pltpu.sync_copy(rows_a, out_hbm.at[pl.ds(t0, TPW)])


@jax.jit
def kernel(x, centroids, w1, w3, w2, bias, scale):
    sc_mesh = plsc.VectorSubcoreMesh(
        core_axis_name="c", subcore_axis_name="s", num_cores=NC)
    n = x.shape[0]
    b2 = bias.reshape(1, E)
    fi, pi, bal, dl, bm, bs, p1, p2, g1, g2, be = pl.pallas_call(
        _router_body,
        out_shape=(
            jax.ShapeDtypeStruct((1, E), jnp.float32),
            jax.ShapeDtypeStruct((1, E), jnp.float32),
            jax.ShapeDtypeStruct((1, 1), jnp.float32),
            jax.ShapeDtypeStruct((1, 1), jnp.float32),
            jax.ShapeDtypeStruct((1, 1), jnp.float32),
            jax.ShapeDtypeStruct((1, 1), jnp.float32),
            jax.ShapeDtypeStruct((n, 1), jnp.int32),
            jax.ShapeDtypeStruct((n, 1), jnp.int32),
            jax.ShapeDtypeStruct((n, 16), jnp.float32),
            jax.ShapeDtypeStruct((n, 16), jnp.float32),
            jax.ShapeDtypeStruct((NB, 1), jnp.int32),
        ),
    )(x, centroids, b2)

    p1f = p1.reshape(n)
    p2f = p2.reshape(n)

    dispatch = pl.kernel(
        _dispatch_body,
        out_type=jax.ShapeDtypeStruct((CAP, D_MODEL), jnp.float32),
        mesh=sc_mesh,
        scratch_types=[
            pltpu.VMEM((TPW,), jnp.int32),
            pltpu.VMEM((TPW,), jnp.int32),
            pltpu.VMEM((TPW, D_MODEL), jnp.float32),
            pltpu.SemaphoreType.DMA,
        ],
    )
    xs = dispatch(x, p1f, p2f)

    eo = pl.pallas_call(
        _expert_body,
        grid_spec=pltpu.PrefetchScalarGridSpec(
            num_scalar_prefetch=1,
            grid=(NB,),
            in_specs=[
                pl.BlockSpec((BLK, D_MODEL), lambda b, be: (b, 0)),
                pl.BlockSpec((E, D_MODEL, D_EXPERT), lambda b, be: (0, 0, 0)),
                pl.BlockSpec((E, D_MODEL, D_EXPERT), lambda b, be: (0, 0, 0)),
                pl.BlockSpec((E, D_EXPERT, D_MODEL), lambda b, be: (0, 0, 0)),
                pl.BlockSpec((1, 1), lambda b, be: (0, 0)),
            ],
            out_specs=pl.BlockSpec((BLK, D_MODEL), lambda b, be: (b, 0)),
        ),
        out_shape=jax.ShapeDtypeStruct((CAP, D_MODEL), jnp.float32),
        compiler_params=pltpu.CompilerParams(
            dimension_semantics=("arbitrary",),
        ),
    )(be.reshape(NB), xs, w1.astype(jnp.bfloat16), w3.astype(jnp.bfloat16),
      w2.astype(jnp.bfloat16), scale.reshape(1, 1))

    combine = pl.kernel(
        _combine_body,
        out_type=jax.ShapeDtypeStruct((n, D_MODEL), jnp.float32),
        mesh=sc_mesh,
        scratch_types=[
            pltpu.VMEM((TPW,), jnp.int32),
            pltpu.VMEM((TPW,), jnp.int32),
            pltpu.VMEM((TPW, 16), jnp.float32),
            pltpu.VMEM((TPW, 16), jnp.float32),
            pltpu.VMEM((TPW, D_MODEL), jnp.float32),
            pltpu.VMEM((TPW, D_MODEL), jnp.float32),
            pltpu.SemaphoreType.DMA,
        ],
    )
    out = combine(eo, p1f, p2f, g1, g2)

    return (out, fi.reshape(E), pi.reshape(E), bal.reshape(()),
            dl.reshape(1), bm.reshape(()), bs.reshape(()))


# speedup vs baseline: 2.1851x; 1.0734x over previous
import jax
import jax.numpy as jnp
from jax import lax
from jax.experimental import pallas as pl
from jax.experimental.pallas import tpu as pltpu
from jax.experimental.pallas import tpu_sc as plsc

TOKENS = 2048
D_MODEL = 768
D_EXPERT = 384
E = 8
TOP_K = 2

BLK = 256
NB = TOKENS * TOP_K // BLK + E
CAP = NB * BLK

NC = 2
NS = 16
NW = NC * NS
TPW = TOKENS // NW
RPW = CAP // NW


def _router_body(x_ref, c_ref, b_ref,
                 fi_ref, pi_ref, bal_ref, dl_ref, bm_ref, bs_ref,
                 p1_ref, p2_ref, g1_ref, g2_ref, be_ref):
    x = x_ref[...]
    c = c_ref[...]
    b = b_ref[...]
    logits = lax.dot_general(
        x, c, (((1,), (1,)), ((), ())),
        preferred_element_type=jnp.float32) + b
    scores = jax.nn.sigmoid(logits)
    e_iota = lax.broadcasted_iota(jnp.int32, scores.shape, 1)
    m1 = jnp.max(scores, axis=1, keepdims=True)
    idx1 = jnp.min(jnp.where(scores == m1, e_iota, E), axis=1, keepdims=True)
    oh1 = (e_iota == idx1).astype(jnp.float32)
    masked = jnp.where(e_iota == idx1, -jnp.inf, scores)
    m2 = jnp.max(masked, axis=1, keepdims=True)
    idx2 = jnp.min(jnp.where(masked == m2, e_iota, E), axis=1, keepdims=True)
    oh2 = (e_iota == idx2).astype(jnp.float32)
    denom = jnp.clip(m1 + m2, 1e-9, None)
    g1 = m1 / denom
    g2 = m2 / denom
    n = x.shape[0]

    sel = oh1 + oh2
    counts = None
    inc = sel
    d = 1
    while d < n:
        inc = inc + jnp.concatenate(
            [jnp.zeros((d, E), jnp.float32), inc[:-d, :]], axis=0)
        d *= 2
    counts = inc[n - 1:n, :]
    fi = counts / (n * TOP_K)
    pi = jnp.sum(g1 * oh1 + g2 * oh2, axis=0, keepdims=True) / n
    fi_ref[...] = fi
    pi_ref[...] = pi
    bal_ref[...] = jnp.sum(fi * pi, keepdims=True).reshape(1, 1)
    dl_ref[...] = jnp.sum(fi, keepdims=True).reshape(1, 1)
    bm = jnp.mean(b)
    bm_ref[...] = bm.reshape(1, 1)
    bs_ref[...] = jnp.sqrt(jnp.sum((b - bm) ** 2) / (E - 1)).reshape(1, 1)

    excl = inc - sel
    counts_i = counts.astype(jnp.int32)
    pc = (((counts_i + (BLK - 1)) // BLK) * BLK).astype(jnp.float32)
    tri = (lax.broadcasted_iota(jnp.int32, (E, E), 0) <
           lax.broadcasted_iota(jnp.int32, (E, E), 1)).astype(jnp.float32)
    off = lax.dot_general(pc, tri, (((1,), (0,)), ((), ())),
                          preferred_element_type=jnp.float32)
    ends = off + pc
    rank1 = jnp.sum(oh1 * excl, axis=1, keepdims=True)
    rank2 = jnp.sum(oh2 * excl, axis=1, keepdims=True)
    base1 = jnp.sum(oh1 * off, axis=1, keepdims=True)
    base2 = jnp.sum(oh2 * off, axis=1, keepdims=True)
    p1_ref[...] = (base1 + rank1).astype(jnp.int32)
    p2_ref[...] = (base2 + rank2).astype(jnp.int32)
    ones128 = jnp.ones((1, 128), jnp.float32)
    g1_ref[...] = g1 * ones128
    g2_ref[...] = g2 * ones128
    bstart = (lax.broadcasted_iota(jnp.int32, (NB, E), 0) * BLK).astype(
        jnp.float32)
    nseg_done = jnp.sum((bstart >= ends).astype(jnp.int32), axis=1,
                        keepdims=True)
    be_ref[...] = jnp.minimum(nseg_done, E - 1)


def _dispatch_body(x_hbm, p1_hbm, p2_hbm, g1_hbm, g2_hbm, xs_hbm, gs_hbm,
                   ia, ib, rows_v, ga_v, gb_v, sem):
    cid = lax.axis_index("c")
    sid = lax.axis_index("s")
    wid = sid * NC + cid
    t0 = wid * TPW
    pltpu.sync_copy(p1_hbm.at[pl.ds(t0, TPW)], ia)
    pltpu.sync_copy(p2_hbm.at[pl.ds(t0, TPW)], ib)
    pltpu.sync_copy(x_hbm.at[pl.ds(t0, TPW)], rows_v)
    pltpu.sync_copy(g1_hbm.at[pl.ds(t0, TPW)], ga_v)
    pltpu.sync_copy(g2_hbm.at[pl.ds(t0, TPW)], gb_v)
    cpa = pltpu.async_copy(rows_v, xs_hbm.at[ia], sem)
    cpb = pltpu.async_copy(rows_v, xs_hbm.at[ib], sem)
    cpc = pltpu.async_copy(ga_v, gs_hbm.at[ia], sem)
    cpd = pltpu.async_copy(gb_v, gs_hbm.at[ib], sem)
    cpa.wait()
    cpb.wait()
    cpc.wait()
    cpd.wait()


def _expert_body(be_ref, xs_ref, gs_ref, w1_ref, w3_ref, w2_ref, scale_ref,
                 eo_ref):
    e = be_ref[pl.program_id(0)]
    xs = xs_ref[...]
    h1 = jnp.dot(xs, w1_ref[e], preferred_element_type=jnp.float32)
    h3 = jnp.dot(xs, w3_ref[e], preferred_element_type=jnp.float32)
    h = h1 * jax.nn.sigmoid(h1) * h3
    eo = jnp.dot(h, w2_ref[e], preferred_element_type=jnp.float32)
    eo_ref[...] = eo * (gs_ref[:, 0:1] * scale_ref[0, 0])


def _combine_body(eo_hbm, p1_hbm, p2_hbm, out_hbm,
                  idx_a, idx_b, rows_a, rows_b, sem):
    cid = lax.axis_index("c")
    sid = lax.axis_index("s")
    wid = sid * NC + cid
    t0 = wid * TPW
    pltpu.sync_copy(p1_hbm.at[pl.ds(t0, TPW)], idx_a)
    pltpu.sync_copy(p2_hbm.at[pl.ds(t0, TPW)], idx_b)
    cpa = pltpu.async_copy(eo_hbm.at[idx_a], rows_a, sem)
    cpb = pltpu.async_copy(eo_hbm.at[idx_b], rows_b, sem)
    cpa.wait()
    cpb.wait()

    def body(i, carry):
        for j in range(D_MODEL // 16):
            s = pl.ds(j * 16, 16)
            rows_a[i, s] = rows_a[i, s] + rows_b[i, s]
        return carry

    lax.fori_loop(0, TPW, body, 0)
    pltpu.sync_copy(rows_a, out_hbm.at[pl.ds(t0, TPW)])


@jax.jit
def kernel(x, centroids, w1, w3, w2, bias, scale):
    sc_mesh = plsc.VectorSubcoreMesh(
        core_axis_name="c", subcore_axis_name="s", num_cores=NC)
    n = x.shape[0]
    b2 = bias.reshape(1, E)
    fi, pi, bal, dl, bm, bs, p1, p2, g1, g2, be = pl.pallas_call(
        _router_body,
        out_shape=(
            jax.ShapeDtypeStruct((1, E), jnp.float32),
            jax.ShapeDtypeStruct((1, E), jnp.float32),
            jax.ShapeDtypeStruct((1, 1), jnp.float32),
            jax.ShapeDtypeStruct((1, 1), jnp.float32),
            jax.ShapeDtypeStruct((1, 1), jnp.float32),
            jax.ShapeDtypeStruct((1, 1), jnp.float32),
            jax.ShapeDtypeStruct((n, 1), jnp.int32),
            jax.ShapeDtypeStruct((n, 1), jnp.int32),
            jax.ShapeDtypeStruct((n, 128), jnp.float32),
            jax.ShapeDtypeStruct((n, 128), jnp.float32),
            jax.ShapeDtypeStruct((NB, 1), jnp.int32),
        ),
    )(x, centroids, b2)

    p1f = p1.reshape(n)
    p2f = p2.reshape(n)

    dispatch = pl.kernel(
        _dispatch_body,
        out_type=(
            jax.ShapeDtypeStruct((CAP, D_MODEL), jnp.float32),
            jax.ShapeDtypeStruct((CAP, 128), jnp.float32),
        ),
        mesh=sc_mesh,
        scratch_types=[
            pltpu.VMEM((TPW,), jnp.int32),
            pltpu.VMEM((TPW,), jnp.int32),
            pltpu.VMEM((TPW, D_MODEL), jnp.float32),
            pltpu.VMEM((TPW, 128), jnp.float32),
            pltpu.VMEM((TPW, 128), jnp.float32),
            pltpu.SemaphoreType.DMA,
        ],
    )
    xs, gs = dispatch(x, p1f, p2f, g1, g2)

    eo = pl.pallas_call(
        _expert_body,
        grid_spec=pltpu.PrefetchScalarGridSpec(
            num_scalar_prefetch=1,
            grid=(NB,),
            in_specs=[
                pl.BlockSpec((BLK, D_MODEL), lambda b, be: (b, 0)),
                pl.BlockSpec((BLK, 128), lambda b, be: (b, 0)),
                pl.BlockSpec((E, D_MODEL, D_EXPERT), lambda b, be: (0, 0, 0)),
                pl.BlockSpec((E, D_MODEL, D_EXPERT), lambda b, be: (0, 0, 0)),
                pl.BlockSpec((E, D_EXPERT, D_MODEL), lambda b, be: (0, 0, 0)),
                pl.BlockSpec((1, 1), lambda b, be: (0, 0)),
            ],
            out_specs=pl.BlockSpec((BLK, D_MODEL), lambda b, be: (b, 0)),
        ),
        out_shape=jax.ShapeDtypeStruct((CAP, D_MODEL), jnp.float32),
        compiler_params=pltpu.CompilerParams(
            dimension_semantics=("arbitrary",),
        ),
    )(be.reshape(NB), xs, gs, w1, w3, w2, scale.reshape(1, 1))

    combine = pl.kernel(
        _combine_body,
        out_type=jax.ShapeDtypeStruct((n, D_MODEL), jnp.float32),
        mesh=sc_mesh,
        scratch_types=[
            pltpu.VMEM((TPW,), jnp.int32),
            pltpu.VMEM((TPW,), jnp.int32),
            pltpu.VMEM((TPW, D_MODEL), jnp.float32),
            pltpu.VMEM((TPW, D_MODEL), jnp.float32),
            pltpu.SemaphoreType.DMA,
        ],
    )
    out = combine(eo, p1f, p2f)

    return (out, fi.reshape(E), pi.reshape(E), bal.reshape(()),
            dl.reshape(1), bm.reshape(()), bs.reshape(()))


# speedup vs baseline: 2.2252x; 1.0184x over previous
import jax
import jax.numpy as jnp
from jax import lax
from jax.experimental import pallas as pl
from jax.experimental.pallas import tpu as pltpu
from jax.experimental.pallas import tpu_sc as plsc

TOKENS = 2048
D_MODEL = 768
D_EXPERT = 384
E = 8
TOP_K = 2

BLK = 256
NB = TOKENS * TOP_K // BLK + E
CAP = NB * BLK

NC = 2
NS = 16
NW = NC * NS
TPW = TOKENS // NW
RPW = CAP // NW


def _router_body(x_ref, c_ref, b_ref,
                 fi_ref, pi_ref, bal_ref, dl_ref, bm_ref, bs_ref,
                 p1_ref, p2_ref, g1_ref, g2_ref, be_ref):
    x = x_ref[...]
    c = c_ref[...]
    b = b_ref[...]
    logits = lax.dot_general(
        x, c, (((1,), (1,)), ((), ())),
        preferred_element_type=jnp.float32) + b
    scores = jax.nn.sigmoid(logits)
    e_iota = lax.broadcasted_iota(jnp.int32, scores.shape, 1)
    m1 = jnp.max(scores, axis=1, keepdims=True)
    idx1 = jnp.min(jnp.where(scores == m1, e_iota, E), axis=1, keepdims=True)
    oh1 = (e_iota == idx1).astype(jnp.float32)
    masked = jnp.where(e_iota == idx1, -jnp.inf, scores)
    m2 = jnp.max(masked, axis=1, keepdims=True)
    idx2 = jnp.min(jnp.where(masked == m2, e_iota, E), axis=1, keepdims=True)
    oh2 = (e_iota == idx2).astype(jnp.float32)
    denom = jnp.clip(m1 + m2, 1e-9, None)
    g1 = m1 / denom
    g2 = m2 / denom
    n = x.shape[0]

    sel = oh1 + oh2
    counts = None
    inc = sel
    d = 1
    while d < n:
        inc = inc + jnp.concatenate(
            [jnp.zeros((d, E), jnp.float32), inc[:-d, :]], axis=0)
        d *= 2
    counts = inc[n - 1:n, :]
    fi = counts / (n * TOP_K)
    pi = jnp.sum(g1 * oh1 + g2 * oh2, axis=0, keepdims=True) / n
    fi_ref[...] = fi
    pi_ref[...] = pi
    bal_ref[...] = jnp.sum(fi * pi, keepdims=True).reshape(1, 1)
    dl_ref[...] = jnp.sum(fi, keepdims=True).reshape(1, 1)
    bm = jnp.mean(b)
    bm_ref[...] = bm.reshape(1, 1)
    bs_ref[...] = jnp.sqrt(jnp.sum((b - bm) ** 2) / (E - 1)).reshape(1, 1)

    excl = inc - sel
    counts_i = counts.astype(jnp.int32)
    pc = (((counts_i + (BLK - 1)) // BLK) * BLK).astype(jnp.float32)
    tri = (lax.broadcasted_iota(jnp.int32, (E, E), 0) <
           lax.broadcasted_iota(jnp.int32, (E, E), 1)).astype(jnp.float32)
    off = lax.dot_general(pc, tri, (((1,), (0,)), ((), ())),
                          preferred_element_type=jnp.float32)
    ends = off + pc
    rank1 = jnp.sum(oh1 * excl, axis=1, keepdims=True)
    rank2 = jnp.sum(oh2 * excl, axis=1, keepdims=True)
    base1 = jnp.sum(oh1 * off, axis=1, keepdims=True)
    base2 = jnp.sum(oh2 * off, axis=1, keepdims=True)
    p1_ref[...] = (base1 + rank1).astype(jnp.int32)
    p2_ref[...] = (base2 + rank2).astype(jnp.int32)
    ones16 = jnp.ones((1, 16), jnp.float32)
    g1_ref[...] = g1 * ones16
    g2_ref[...] = g2 * ones16
    bstart = (lax.broadcasted_iota(jnp.int32, (NB, E), 0) * BLK).astype(
        jnp.float32)
    nseg_done = jnp.sum((bstart >= ends).astype(jnp.int32), axis=1,
                        keepdims=True)
    be_ref[...] = jnp.minimum(nseg_done, E - 1)


def _dispatch_body(x_hbm, p1_hbm, p2_hbm, xs_hbm,
                   ia, ib, rows_v, sem):
    cid = lax.axis_index("c")
    sid = lax.axis_index("s")
    wid = sid * NC + cid
    t0 = wid * TPW
    pltpu.sync_copy(p1_hbm.at[pl.ds(t0, TPW)], ia)
    pltpu.sync_copy(p2_hbm.at[pl.ds(t0, TPW)], ib)
    pltpu.sync_copy(x_hbm.at[pl.ds(t0, TPW)], rows_v)
    cpa = pltpu.async_copy(rows_v, xs_hbm.at[ia], sem)
    cpb = pltpu.async_copy(rows_v, xs_hbm.at[ib], sem)
    cpa.wait()
    cpb.wait()


def _expert_body(be_ref, xs_ref, w1_ref, w3_ref, w2_ref, scale_ref,
                 eo_ref):
    e = be_ref[pl.program_id(0)]
    xs = xs_ref[...]
    h1 = jnp.dot(xs, w1_ref[e], preferred_element_type=jnp.float32)
    h3 = jnp.dot(xs, w3_ref[e], preferred_element_type=jnp.float32)
    h = h1 * jax.nn.sigmoid(h1) * h3
    eo = jnp.dot(h, w2_ref[e], preferred_element_type=jnp.float32)
    eo_ref[...] = eo * scale_ref[0, 0]


def _combine_body(eo_hbm, p1_hbm, p2_hbm, g1_hbm, g2_hbm, out_hbm,
                  idx_a, idx_b, ga, gb, rows_a, rows_b, sem):
    cid = lax.axis_index("c")
    sid = lax.axis_index("s")
    wid = sid * NC + cid
    t0 = wid * TPW
    pltpu.sync_copy(p1_hbm.at[pl.ds(t0, TPW)], idx_a)
    pltpu.sync_copy(p2_hbm.at[pl.ds(t0, TPW)], idx_b)
    cpa = pltpu.async_copy(eo_hbm.at[idx_a], rows_a, sem)
    cpb = pltpu.async_copy(eo_hbm.at[idx_b], rows_b, sem)
    pltpu.sync_copy(g1_hbm.at[pl.ds(t0, TPW)], ga)
    pltpu.sync_copy(g2_hbm.at[pl.ds(t0, TPW)], gb)
    cpa.wait()
    cpb.wait()

    def body(i, carry):
        gav = ga[i]
        gbv = gb[i]
        for j in range(D_MODEL // 16):
            s = pl.ds(j * 16, 16)
            rows_a[i, s] = gav * rows_a[i, s] + gbv * rows_b[i, s]
        return carry

    lax.fori_loop(0, TPW, body, 0)
    pltpu.sync_copy(rows_a, out_hbm.at[pl.ds(t0, TPW)])


@jax.jit
def kernel(x, centroids, w1, w3, w2, bias, scale):
    sc_mesh = plsc.VectorSubcoreMesh(
        core_axis_name="c", subcore_axis_name="s", num_cores=NC)
    n = x.shape[0]
    b2 = bias.reshape(1, E)
    fi, pi, bal, dl, bm, bs, p1, p2, g1, g2, be = pl.pallas_call(
        _router_body,
        out_shape=(
            jax.ShapeDtypeStruct((1, E), jnp.float32),
            jax.ShapeDtypeStruct((1, E), jnp.float32),
            jax.ShapeDtypeStruct((1, 1), jnp.float32),
            jax.ShapeDtypeStruct((1, 1), jnp.float32),
            jax.ShapeDtypeStruct((1, 1), jnp.float32),
            jax.ShapeDtypeStruct((1, 1), jnp.float32),
            jax.ShapeDtypeStruct((n, 1), jnp.int32),
            jax.ShapeDtypeStruct((n, 1), jnp.int32),
            jax.ShapeDtypeStruct((n, 16), jnp.float32),
            jax.ShapeDtypeStruct((n, 16), jnp.float32),
            jax.ShapeDtypeStruct((NB, 1), jnp.int32),
        ),
    )(x, centroids, b2)

    p1f = p1.reshape(n)
    p2f = p2.reshape(n)

    dispatch = pl.kernel(
        _dispatch_body,
        out_type=jax.ShapeDtypeStruct((CAP, D_MODEL), jnp.float32),
        mesh=sc_mesh,
        scratch_types=[
            pltpu.VMEM((TPW,), jnp.int32),
            pltpu.VMEM((TPW,), jnp.int32),
            pltpu.VMEM((TPW, D_MODEL), jnp.float32),
            pltpu.SemaphoreType.DMA,
        ],
    )
    xs = dispatch(x, p1f, p2f)

    eo = pl.pallas_call(
        _expert_body,
        grid_spec=pltpu.PrefetchScalarGridSpec(
            num_scalar_prefetch=1,
            grid=(NB,),
            in_specs=[
                pl.BlockSpec((BLK, D_MODEL), lambda b, be: (b, 0)),
                pl.BlockSpec((E, D_MODEL, D_EXPERT), lambda b, be: (0, 0, 0)),
                pl.BlockSpec((E, D_MODEL, D_EXPERT), lambda b, be: (0, 0, 0)),
                pl.BlockSpec((E, D_EXPERT, D_MODEL), lambda b, be: (0, 0, 0)),
                pl.BlockSpec((1, 1), lambda b, be: (0, 0)),
            ],
            out_specs=pl.BlockSpec((BLK, D_MODEL), lambda b, be: (b, 0)),
        ),
        out_shape=jax.ShapeDtypeStruct((CAP, D_MODEL), jnp.float32),
        compiler_params=pltpu.CompilerParams(
            dimension_semantics=("arbitrary",),
        ),
    )(be.reshape(NB), xs, w1, w3, w2, scale.reshape(1, 1))

    combine = pl.kernel(
        _combine_body,
        out_type=jax.ShapeDtypeStruct((n, D_MODEL), jnp.float32),
        mesh=sc_mesh,
        scratch_types=[
            pltpu.VMEM((TPW,), jnp.int32),
            pltpu.VMEM((TPW,), jnp.int32),
            pltpu.VMEM((TPW, 16), jnp.float32),
            pltpu.VMEM((TPW, 16), jnp.float32),
            pltpu.VMEM((TPW, D_MODEL), jnp.float32),
            pltpu.VMEM((TPW, D_MODEL), jnp.float32),
            pltpu.SemaphoreType.DMA,
        ],
    )
    out = combine(eo, p1f, p2f, g1, g2)

    return (out, fi.reshape(E), pi.reshape(E), bal.reshape(()),
            dl.reshape(1), bm.reshape(()), bs.reshape(()))
